# baseline (speedup 1.0000x reference)
import functools

import jax
import jax.numpy as jnp
from jax import lax
from jax.experimental import pallas as pl
from jax.experimental.pallas import tpu as pltpu
from jax.experimental.pallas import tpu_sc as plsc

_N = 50000
_D = 128
_S_A = 8
_S_S = 4
_NC = 2
_NS = 16
_NW = _NC * _NS
_N_PAD = 53248


@functools.lru_cache(maxsize=None)
def _make_sc_gat(S):
    dpc = 128 // S
    dwork = _N_PAD // _NW
    nch = dwork * S // 128
    nhalf = dpc // 16
    obr = 2 * dpc
    assert nch % 2 == 0

    mesh = plsc.VectorSubcoreMesh(
        core_axis_name="c", subcore_axis_name="s",
        num_cores=_NC, num_subcores=_NS)

    @functools.partial(
        pl.kernel,
        mesh=mesh,
        compiler_params=pltpu.CompilerParams(needs_layout_passes=False),
        out_type=jax.ShapeDtypeStruct((_N_PAD, _D), jnp.float32),
        scratch_types=[
            pltpu.VMEM((_N_PAD,), jnp.float32),
            pltpu.VMEM((dwork,), jnp.float32),
            pltpu.VMEM((nch, 128), jnp.int32),
            pltpu.VMEM((nch * 128,), jnp.int32),
            pltpu.VMEM((128, _D), jnp.float32),
            pltpu.VMEM((128, _D), jnp.float32),
            pltpu.VMEM((obr, _D), jnp.float32),
            pltpu.VMEM((256,), jnp.float32),
            pltpu.VMEM((128,), jnp.float32),
            pltpu.SemaphoreType.DMA,
            pltpu.SemaphoreType.DMA,
        ],
    )
    def gat(table, el1, er1, idx3, idxf, bias, out,
            el_v, er_v, idx_v, idxf_v, st0, st1, outbuf, alpha_v, b_v,
            sem0, sem1):
        wid = lax.axis_index("s") * _NC + lax.axis_index("c")
        pltpu.sync_copy(el1, el_v)
        pltpu.sync_copy(er1.at[pl.ds(wid * dwork, dwork)], er_v)
        pltpu.sync_copy(idx3.at[wid], idx_v)
        pltpu.sync_copy(idxf.at[pl.ds(wid * nch * 128, nch * 128)], idxf_v)
        pltpu.sync_copy(bias, b_v)
        stages = (st0, st1)
        sems = (sem0, sem1)
        iota = lax.iota(jnp.int32, 16)

        def issue(c, p):
            return pltpu.async_copy(
                table.at[idx_v.at[c]], stages[p], sems[p])

        def compute_chunk(c, p):
            st = stages[p]
            for h in range(nhalf):
                t = c * dpc + h * 16 + iota
                erh = plsc.load_gather(er_v, [t])
                e_js = []
                for j in range(S):
                    pos = c * 128 + (h * 16) * S + iota * S + j
                    nbr = plsc.load_gather(idxf_v, [pos])
                    elj = plsc.load_gather(el_v, [nbr])
                    e = elj + erh
                    e_js.append(jnp.where(e >= 0.0, e, 0.2 * e))
                m = functools.reduce(jnp.maximum, e_js)
                p_js = [jnp.exp(e - m) for e in e_js]
                rinv = 1.0 / functools.reduce(jnp.add, p_js)
                for j in range(S):
                    pos = 128 + (h * 16) * S + iota * S + j
                    plsc.store_scatter(alpha_v, [pos], p_js[j] * rinv)
            for i in range(dpc):
                accs = [b_v[pl.ds(d * 16, 16)] for d in range(8)]
                for j in range(S):
                    a = plsc.load_gather(
                        alpha_v, [jnp.full((16,), 128 + i * S + j, jnp.int32)])
                    r = i * S + j
                    for d in range(8):
                        accs[d] = accs[d] + a * st[r, pl.ds(d * 16, 16)]
                for d in range(8):
                    outbuf[p * dpc + i, pl.ds(d * 16, 16)] = accs[d]

        def body(k, carry):
            for p in range(2):
                c = 2 * k + p
                issue(c, p).wait()
                compute_chunk(c, p)

            pltpu.sync_copy(
                outbuf, out.at[pl.ds(wid * dwork + k * obr, obr)])
            return carry

        lax.fori_loop(0, nch // 2, body, 0)

    return gat


def _proj_kernel(ha_ref, hs_ref, hp_ref,
                 wa_ref, ws_ref, wda_ref, wds_ref,
                 ala_ref, als_ref, ara_ref, ars_ref,
                 fa_ref, fs_ref, ela_ref, els_ref, era_ref, ers_ref):
    dn = (((1,), (1,)), ((), ()))
    fa = lax.dot_general(ha_ref[...], wa_ref[...], dn,
                         preferred_element_type=jnp.float32)
    fs = lax.dot_general(hs_ref[...], ws_ref[...], dn,
                         preferred_element_type=jnp.float32)
    fa_ref[...] = fa
    fs_ref[...] = fs
    ela_ref[...] = jnp.sum(
        fa.reshape(8, 128, _D) * ala_ref[...].reshape(1, 1, _D), axis=-1)
    els_ref[...] = jnp.sum(
        fs.reshape(8, 128, _D) * als_ref[...].reshape(1, 1, _D), axis=-1)
    dn2 = (((1,), (0,)), ((), ()))
    wva = lax.dot_general(ara_ref[...], wda_ref[...], dn2,
                          preferred_element_type=jnp.float32)
    wvs = lax.dot_general(ars_ref[...], wds_ref[...], dn2,
                          preferred_element_type=jnp.float32)
    hp3 = hp_ref[...].reshape(8, 128, _D)
    era_ref[...] = jnp.sum(hp3 * wva.reshape(1, 1, _D), axis=-1)
    ers_ref[...] = jnp.sum(hp3 * wvs.reshape(1, 1, _D), axis=-1)


def _beta_kernel(e0_ref, e1_ref, wfc_ref, bfc_ref, att_ref, l0_ref, l1_ref):
    @pl.when(pl.program_id(0) == 0)
    def _():
        l0_ref[0, 0] = 0.0
        l1_ref[0, 0] = 0.0

    dn = (((1,), (1,)), ((), ()))

    def part(e):
        t = jnp.tanh(
            lax.dot_general(e, wfc_ref[...], dn,
                            preferred_element_type=jnp.float32)
            + bfc_ref[...])
        return jnp.sum(t * att_ref[...])

    l0_ref[0, 0] += part(e0_ref[...])
    l1_ref[0, 0] += part(e1_ref[...])


def _combine_kernel(l0_ref, l1_ref, e0_ref, e1_ref, z_ref):
    d = (l1_ref[0, 0] - l0_ref[0, 0]) * (1.0 / _N)
    e0 = e0_ref[...]
    beta0 = 1.0 / (1.0 + jnp.exp(jnp.full(e0.shape, d, jnp.float32)))
    z_ref[...] = beta0 * e0 + (1.0 - beta0) * e1_ref[...]


def _flat_idx(nbr, S):
    nbr = nbr.astype(jnp.int32)
    pad = jnp.zeros((_N_PAD - _N, S), jnp.int32)
    return jnp.concatenate([nbr, pad], axis=0).reshape(-1)


def kernel(h_paper, h_author, h_subject,
           Wsrc_a, Wdst_a, al_a, ar_a, b_a,
           Wsrc_s, Wdst_s, al_s, ar_s, b_s,
           W_fc, b_fc, att,
           nbr_author, nbr_subject):
    pad = ((0, _N_PAD - _N), (0, 0))
    ha = jnp.pad(h_author, pad)
    hs = jnp.pad(h_subject, pad)
    hp = jnp.pad(h_paper, pad)

    BN1 = 1024
    fullmat = pl.BlockSpec((_D, _D), lambda i: (0, 0))
    vec2 = pl.BlockSpec((1, _D), lambda i: (0, 0))
    hblk = pl.BlockSpec((BN1, _D), lambda i: (i, 0))
    eblk = pl.BlockSpec((8, 128), lambda i: (i, 0))
    erows = _N_PAD // 128
    fs_a, fs_s, el_a, el_s, er_a, er_s = pl.pallas_call(
        _proj_kernel,
        grid=(_N_PAD // BN1,),
        in_specs=[hblk, hblk, hblk, fullmat, fullmat, fullmat, fullmat,
                  vec2, vec2, vec2, vec2],
        out_specs=[hblk, hblk, eblk, eblk, eblk, eblk],
        out_shape=[jax.ShapeDtypeStruct((_N_PAD, _D), jnp.float32)] * 2
        + [jax.ShapeDtypeStruct((erows, 128), jnp.float32)] * 4,
    )(ha, hs, hp, Wsrc_a, Wsrc_s, Wdst_a, Wdst_s,
      al_a.reshape(1, _D), al_s.reshape(1, _D),
      ar_a.reshape(1, _D), ar_s.reshape(1, _D))

    idx_a = _flat_idx(nbr_author, _S_A)
    idx_s = _flat_idx(nbr_subject, _S_S)
    e0 = _make_sc_gat(_S_A)(fs_a, el_a.reshape(-1), er_a.reshape(-1),
                            idx_a.reshape(_NW, -1, 128), idx_a, b_a)
    e1 = _make_sc_gat(_S_S)(fs_s, el_s.reshape(-1), er_s.reshape(-1),
                            idx_s.reshape(_NW, -1, 128), idx_s, b_s)

    BN = 400
    grid = (_N // BN,)
    nblk = pl.BlockSpec((BN, _D), lambda i: (i, 0))

    l0, l1 = pl.pallas_call(
        _beta_kernel,
        grid=grid,
        in_specs=[nblk, nblk, fullmat, vec2, vec2],
        out_specs=[pl.BlockSpec(memory_space=pltpu.SMEM)] * 2,
        out_shape=[jax.ShapeDtypeStruct((1, 1), jnp.float32)] * 2,
    )(e0, e1, W_fc, b_fc.reshape(1, _D), att.reshape(1, _D))

    z = pl.pallas_call(
        _combine_kernel,
        grid=grid,
        in_specs=[pl.BlockSpec(memory_space=pltpu.SMEM)] * 2 + [nblk, nblk],
        out_specs=nblk,
        out_shape=jax.ShapeDtypeStruct((_N, _D), jnp.float32),
    )(l0, l1, e0, e1)

    return z


# speedup vs baseline: 1.0116x; 1.0116x over previous
import functools

import jax
import jax.numpy as jnp
from jax import lax
from jax.experimental import pallas as pl
from jax.experimental.pallas import tpu as pltpu
from jax.experimental.pallas import tpu_sc as plsc

_N = 50000
_D = 128
_S_A = 8
_S_S = 4
_NC = 2
_NS = 16
_NW = _NC * _NS
_N_PAD = 53248


@functools.lru_cache(maxsize=None)
def _make_sc_gat(S):
    dpc = 128 // S
    dwork = _N_PAD // _NW
    nch = dwork * S // 128
    nhalf = dpc // 16
    obr = 2 * dpc
    assert nch % 2 == 0

    mesh = plsc.VectorSubcoreMesh(
        core_axis_name="c", subcore_axis_name="s",
        num_cores=_NC, num_subcores=_NS)

    @functools.partial(
        pl.kernel,
        mesh=mesh,
        compiler_params=pltpu.CompilerParams(needs_layout_passes=False),
        out_type=jax.ShapeDtypeStruct((_N_PAD, _D), jnp.float32),
        scratch_types=[
            pltpu.VMEM((_N_PAD,), jnp.float32),
            pltpu.VMEM((dwork,), jnp.float32),
            pltpu.VMEM((nch, 128), jnp.int32),
            pltpu.VMEM((nch * 128,), jnp.int32),
            pltpu.VMEM((128, _D), jnp.float32),
            pltpu.VMEM((128, _D), jnp.float32),
            pltpu.VMEM((obr, _D), jnp.float32),
            pltpu.VMEM((256,), jnp.float32),
            pltpu.VMEM((128,), jnp.float32),
            pltpu.SemaphoreType.DMA,
            pltpu.SemaphoreType.DMA,
        ],
    )
    def gat(table, el1, er1, idx3, idxf, bias, out,
            el_v, er_v, idx_v, idxf_v, st0, st1, outbuf, alpha_v, b_v,
            sem0, sem1):
        wid = lax.axis_index("s") * _NC + lax.axis_index("c")
        pltpu.sync_copy(el1, el_v)
        pltpu.sync_copy(er1.at[pl.ds(wid * dwork, dwork)], er_v)
        pltpu.sync_copy(idx3.at[wid], idx_v)
        pltpu.sync_copy(idxf.at[pl.ds(wid * nch * 128, nch * 128)], idxf_v)
        pltpu.sync_copy(bias, b_v)
        stages = (st0, st1)
        sems = (sem0, sem1)
        iota = lax.iota(jnp.int32, 16)

        def issue(c, p):
            return pltpu.async_copy(
                table.at[idx_v.at[c]], stages[p], sems[p])

        issue(0, 0)
        issue(1, 1)

        def compute_chunk(c, p):
            st = stages[p]
            for h in range(nhalf):
                t = c * dpc + h * 16 + iota
                erh = plsc.load_gather(er_v, [t])
                e_js = []
                for j in range(S):
                    pos = c * 128 + (h * 16) * S + iota * S + j
                    nbr = plsc.load_gather(idxf_v, [pos])
                    elj = plsc.load_gather(el_v, [nbr])
                    e = elj + erh
                    e_js.append(jnp.where(e >= 0.0, e, 0.2 * e))
                m = functools.reduce(jnp.maximum, e_js)
                p_js = [jnp.exp(e - m) for e in e_js]
                rinv = 1.0 / functools.reduce(jnp.add, p_js)
                for j in range(S):
                    pos = 128 + (h * 16) * S + iota * S + j
                    plsc.store_scatter(alpha_v, [pos], p_js[j] * rinv)
            for i in range(dpc):
                accs = [b_v[pl.ds(d * 16, 16)] for d in range(8)]
                for j in range(S):
                    a = plsc.load_gather(
                        alpha_v, [jnp.full((16,), 128 + i * S + j, jnp.int32)])
                    r = i * S + j
                    for d in range(8):
                        accs[d] = accs[d] + a * st[r, pl.ds(d * 16, 16)]
                for d in range(8):
                    outbuf[p * dpc + i, pl.ds(d * 16, 16)] = accs[d]

        def body(k, carry):
            for p in range(2):
                c = 2 * k + p
                pltpu.make_async_copy(
                    table.at[pl.ds(0, 128)], stages[p], sems[p]).wait()
                compute_chunk(c, p)

                @pl.when(c + 2 < nch)
                def _():
                    issue(c + 2, p)

            pltpu.sync_copy(
                outbuf, out.at[pl.ds(wid * dwork + k * obr, obr)])
            return carry

        lax.fori_loop(0, nch // 2, body, 0)

    return gat


def _proj_kernel(ha_ref, hs_ref, hp_ref,
                 wa_ref, ws_ref, wda_ref, wds_ref,
                 ala_ref, als_ref, ara_ref, ars_ref,
                 fa_ref, fs_ref, ela_ref, els_ref, era_ref, ers_ref):
    dn = (((1,), (1,)), ((), ()))
    fa = lax.dot_general(ha_ref[...], wa_ref[...], dn,
                         preferred_element_type=jnp.float32)
    fs = lax.dot_general(hs_ref[...], ws_ref[...], dn,
                         preferred_element_type=jnp.float32)
    fa_ref[...] = fa
    fs_ref[...] = fs
    ela_ref[...] = jnp.sum(
        fa.reshape(8, 128, _D) * ala_ref[...].reshape(1, 1, _D), axis=-1)
    els_ref[...] = jnp.sum(
        fs.reshape(8, 128, _D) * als_ref[...].reshape(1, 1, _D), axis=-1)
    dn2 = (((1,), (0,)), ((), ()))
    wva = lax.dot_general(ara_ref[...], wda_ref[...], dn2,
                          preferred_element_type=jnp.float32)
    wvs = lax.dot_general(ars_ref[...], wds_ref[...], dn2,
                          preferred_element_type=jnp.float32)
    hp3 = hp_ref[...].reshape(8, 128, _D)
    era_ref[...] = jnp.sum(hp3 * wva.reshape(1, 1, _D), axis=-1)
    ers_ref[...] = jnp.sum(hp3 * wvs.reshape(1, 1, _D), axis=-1)


def _beta_kernel(e0_ref, e1_ref, wfc_ref, bfc_ref, att_ref, l0_ref, l1_ref):
    @pl.when(pl.program_id(0) == 0)
    def _():
        l0_ref[0, 0] = 0.0
        l1_ref[0, 0] = 0.0

    dn = (((1,), (1,)), ((), ()))

    def part(e):
        t = jnp.tanh(
            lax.dot_general(e, wfc_ref[...], dn,
                            preferred_element_type=jnp.float32)
            + bfc_ref[...])
        return jnp.sum(t * att_ref[...])

    l0_ref[0, 0] += part(e0_ref[...])
    l1_ref[0, 0] += part(e1_ref[...])


def _combine_kernel(l0_ref, l1_ref, e0_ref, e1_ref, z_ref):
    d = (l1_ref[0, 0] - l0_ref[0, 0]) * (1.0 / _N)
    e0 = e0_ref[...]
    beta0 = 1.0 / (1.0 + jnp.exp(jnp.full(e0.shape, d, jnp.float32)))
    z_ref[...] = beta0 * e0 + (1.0 - beta0) * e1_ref[...]


def _flat_idx(nbr, S):
    nbr = nbr.astype(jnp.int32)
    pad = jnp.zeros((_N_PAD - _N, S), jnp.int32)
    return jnp.concatenate([nbr, pad], axis=0).reshape(-1)


def kernel(h_paper, h_author, h_subject,
           Wsrc_a, Wdst_a, al_a, ar_a, b_a,
           Wsrc_s, Wdst_s, al_s, ar_s, b_s,
           W_fc, b_fc, att,
           nbr_author, nbr_subject):
    pad = ((0, _N_PAD - _N), (0, 0))
    ha = jnp.pad(h_author, pad)
    hs = jnp.pad(h_subject, pad)
    hp = jnp.pad(h_paper, pad)

    BN1 = 1024
    fullmat = pl.BlockSpec((_D, _D), lambda i: (0, 0))
    vec2 = pl.BlockSpec((1, _D), lambda i: (0, 0))
    hblk = pl.BlockSpec((BN1, _D), lambda i: (i, 0))
    eblk = pl.BlockSpec((8, 128), lambda i: (i, 0))
    erows = _N_PAD // 128
    fs_a, fs_s, el_a, el_s, er_a, er_s = pl.pallas_call(
        _proj_kernel,
        grid=(_N_PAD // BN1,),
        in_specs=[hblk, hblk, hblk, fullmat, fullmat, fullmat, fullmat,
                  vec2, vec2, vec2, vec2],
        out_specs=[hblk, hblk, eblk, eblk, eblk, eblk],
        out_shape=[jax.ShapeDtypeStruct((_N_PAD, _D), jnp.float32)] * 2
        + [jax.ShapeDtypeStruct((erows, 128), jnp.float32)] * 4,
    )(ha, hs, hp, Wsrc_a, Wsrc_s, Wdst_a, Wdst_s,
      al_a.reshape(1, _D), al_s.reshape(1, _D),
      ar_a.reshape(1, _D), ar_s.reshape(1, _D))

    idx_a = _flat_idx(nbr_author, _S_A)
    idx_s = _flat_idx(nbr_subject, _S_S)
    e0 = _make_sc_gat(_S_A)(fs_a, el_a.reshape(-1), er_a.reshape(-1),
                            idx_a.reshape(_NW, -1, 128), idx_a, b_a)
    e1 = _make_sc_gat(_S_S)(fs_s, el_s.reshape(-1), er_s.reshape(-1),
                            idx_s.reshape(_NW, -1, 128), idx_s, b_s)

    BN = 400
    grid = (_N // BN,)
    nblk = pl.BlockSpec((BN, _D), lambda i: (i, 0))

    l0, l1 = pl.pallas_call(
        _beta_kernel,
        grid=grid,
        in_specs=[nblk, nblk, fullmat, vec2, vec2],
        out_specs=[pl.BlockSpec(memory_space=pltpu.SMEM)] * 2,
        out_shape=[jax.ShapeDtypeStruct((1, 1), jnp.float32)] * 2,
    )(e0, e1, W_fc, b_fc.reshape(1, _D), att.reshape(1, _D))

    z = pl.pallas_call(
        _combine_kernel,
        grid=grid,
        in_specs=[pl.BlockSpec(memory_space=pltpu.SMEM)] * 2 + [nblk, nblk],
        out_specs=nblk,
        out_shape=jax.ShapeDtypeStruct((_N, _D), jnp.float32),
    )(l0, l1, e0, e1)

    return z
